# baseline (speedup 1.0000x reference)
import functools

import jax
import jax.numpy as jnp
from jax import lax
from jax.experimental import pallas as pl
from jax.experimental.pallas import tpu as pltpu
from jax.experimental.pallas import tpu_sc as plsc

_PERMUTATION_COUNT = 2


def _tc_lengths(L, B):
    def body(m_ref, o_ref):
        ones = jnp.ones((8, L), jnp.float32)
        cnt = jax.lax.dot(ones, m_ref[...].astype(jnp.float32),
                          preferred_element_type=jnp.float32)
        o_ref[...] = cnt.astype(jnp.int32)

    return pl.pallas_call(
        body,
        out_shape=jax.ShapeDtypeStruct((8, B), jnp.int32),
    )


def _make_sc_kernel(B, L, D, O, NC, NS):
    NW = NC * NS
    rpw = O // NW
    s_l = (D // 8) * (B // 128) * 8 * 128
    s_dt = (B // 128) * 8 * 128
    s_bt = 8 * 128

    mesh = plsc.VectorSubcoreMesh(core_axis_name="c", subcore_axis_name="s")

    gdnums = lax.GatherDimensionNumbers(
        offset_dims=(), collapsed_slice_dims=(0,), start_index_map=(0,)
    )

    def _lane_gather(v, idx16):
        return lax.gather(
            v, idx16[:, None], gdnums, slice_sizes=(1,),
            mode=lax.GatherScatterMode.PROMISE_IN_BOUNDS,
        )

    @functools.partial(
        pl.kernel,
        out_type=jax.ShapeDtypeStruct((D // 8, O // 16, 128), jnp.float32),
        mesh=mesh,
        scratch_types=[
            pltpu.VMEM((128,), jnp.int32),
            pltpu.VMEM((D, rpw), jnp.int32),
            pltpu.VMEM((D // 8, 8 * 128), jnp.float32),
            pltpu.VMEM((D // 8, 8, rpw), jnp.float32),
            pltpu.SemaphoreType.DMA,
        ],
        compiler_params=pltpu.CompilerParams(
            use_tc_tiling_on_sc=False, needs_layout_passes=False
        ),
    )
    def sc_kernel(cnt_hbm, emb_hbm, out_hbm,
                  cnt_v, idx_v, blk_v, dst_v, sem):
        wid = lax.axis_index("s") * NC + lax.axis_index("c")
        pltpu.sync_copy(cnt_hbm.at[pl.ds(wid * 1024, 128)], cnt_v)

        lane = lax.iota(jnp.int32, 16)
        cnts = [cnt_v[pl.ds(k * 16, 16)] for k in range(8)]

        lbs, lvs = [], []
        for g in range(4):
            v = jnp.zeros((16,), jnp.int32)
            for r in range(16):
                bc = _PERMUTATION_COUNT * (g * 16 + r)
                v = jnp.where(lane == r, cnts[bc // 16][bc % 16], v)
            l_idx = jnp.maximum(v, 1) - 1
            lvs.append(l_idx)
            lbs.append(l_idx * s_l + wid * s_bt
                       + _PERMUTATION_COUNT * (g * 16) + _PERMUTATION_COUNT * lane)

        lmin = jnp.min(jnp.minimum(jnp.minimum(lvs[0], lvs[1]),
                                   jnp.minimum(lvs[2], lvs[3])))
        lmax = jnp.max(jnp.maximum(jnp.maximum(lvs[0], lvs[1]),
                                   jnp.maximum(lvs[2], lvs[3])))
        uniform = lmin == lmax

        @pl.when(uniform)
        def _fast():
            base = lmin * s_l + wid * s_bt
            for dt in range(D // 8):
                pltpu.sync_copy(
                    emb_hbm.at[pl.ds(base + dt * s_dt, 8 * 128)],
                    blk_v.at[dt],
                )
            ev = jnp.where(lane < 8, lane * 2, lane * 2 - 16)

            def dt_body(dt, _):
                for dr in range(8):
                    vs = [blk_v[dt, pl.ds(dr * 128 + k * 16, 16)]
                          for k in range(8)]
                    for k in range(4):
                        g1 = _lane_gather(vs[2 * k], ev)
                        g2 = _lane_gather(vs[2 * k + 1], ev)
                        dst_v[dt, dr, pl.ds(k * 16, 16)] = (
                            jnp.where(lane < 8, g1, g2))
                return 0

            lax.fori_loop(0, D // 8, dt_body, 0)

        @pl.when(jnp.logical_not(uniform))
        def _slow():
            def idx_body(d, _):
                hi = (d >> 3) * s_dt + (d & 7) * 128
                for g in range(4):
                    idx_v[d, pl.ds(g * 16, 16)] = lbs[g] + hi
                return 0

            lax.fori_loop(0, D, idx_body, 0)

            def fire_body(d, _):
                pltpu.async_copy(
                    emb_hbm.at[idx_v.at[d]], dst_v.at[d >> 3, d & 7], sem
                )
                return 0

            def drain_body(d, _):
                pltpu.make_async_copy(
                    emb_hbm.at[idx_v.at[d]], dst_v.at[d >> 3, d & 7], sem
                ).wait()
                return 0

            lax.fori_loop(0, D, fire_body, 0)
            lax.fori_loop(0, D, drain_body, 0)
        pltpu.sync_copy(
            dst_v,
            out_hbm.at[:, pl.ds((wid >> 1) * 8, 8),
                       pl.ds((wid & 1) * rpw, rpw)],
        )

    return sc_kernel


def kernel(embeddings, labels, embeddings_mask, labels_mask):
    B, L, D = embeddings.shape
    O = len(range(0, B - 1, _PERMUTATION_COUNT))
    info = plsc.get_sparse_core_info()
    NC, NS = info.num_cores, info.num_subcores

    emb_phys = (
        embeddings.reshape(B // 128, 128, L, D // 8, 8)
        .transpose(2, 3, 0, 4, 1)
        .reshape(B * L * D)
    )
    cnt = (
        _tc_lengths(L, B)(labels_mask.T.view(jnp.int8))
        .reshape(8, B // 128, 128)
        .transpose(1, 0, 2)
        .reshape(8 * B)
    )

    out = _make_sc_kernel(B, L, D, O, NC, NS)(cnt, emb_phys)
    return (
        out.reshape(D // 8, O // 128, 8, 128)
        .transpose(1, 3, 0, 2)
        .reshape(O, 1, D)
    )


# speedup vs baseline: 1.0019x; 1.0019x over previous
import functools

import jax
import jax.numpy as jnp
from jax import lax
from jax.experimental import pallas as pl
from jax.experimental.pallas import tpu as pltpu
from jax.experimental.pallas import tpu_sc as plsc

_PERMUTATION_COUNT = 2


def _tc_lengths(L, B):
    def body(m_ref, o_ref):
        cnt = jnp.sum(m_ref[...].astype(jnp.int32), axis=0, keepdims=True)
        o_ref[...] = jnp.broadcast_to(cnt, (8, B))

    return pl.pallas_call(
        body,
        out_shape=jax.ShapeDtypeStruct((8, B), jnp.int32),
    )


def _make_sc_kernel(B, L, D, O, NC, NS):
    NW = NC * NS
    rpw = O // NW
    s_l = (D // 8) * (B // 128) * 8 * 128
    s_dt = (B // 128) * 8 * 128
    s_bt = 8 * 128

    mesh = plsc.VectorSubcoreMesh(core_axis_name="c", subcore_axis_name="s")

    gdnums = lax.GatherDimensionNumbers(
        offset_dims=(), collapsed_slice_dims=(0,), start_index_map=(0,)
    )

    def _lane_gather(v, idx16):
        return lax.gather(
            v, idx16[:, None], gdnums, slice_sizes=(1,),
            mode=lax.GatherScatterMode.PROMISE_IN_BOUNDS,
        )

    @functools.partial(
        pl.kernel,
        out_type=jax.ShapeDtypeStruct((D // 8, O // 16, 128), jnp.float32),
        mesh=mesh,
        scratch_types=[
            pltpu.VMEM((128,), jnp.int32),
            pltpu.VMEM((D, rpw), jnp.int32),
            pltpu.VMEM((D // 8, 8 * 128), jnp.float32),
            pltpu.VMEM((D // 8, 8, rpw), jnp.float32),
            pltpu.SemaphoreType.DMA,
        ],
        compiler_params=pltpu.CompilerParams(
            use_tc_tiling_on_sc=False, needs_layout_passes=False
        ),
    )
    def sc_kernel(cnt_hbm, emb_hbm, out_hbm,
                  cnt_v, idx_v, blk_v, dst_v, sem):
        wid = lax.axis_index("s") * NC + lax.axis_index("c")
        pltpu.sync_copy(cnt_hbm.at[pl.ds(wid * 1024, 128)], cnt_v)

        lane = lax.iota(jnp.int32, 16)
        cnts = [cnt_v[pl.ds(k * 16, 16)] for k in range(8)]

        lbs, lvs = [], []
        for g in range(4):
            v = jnp.zeros((16,), jnp.int32)
            for r in range(16):
                bc = _PERMUTATION_COUNT * (g * 16 + r)
                v = jnp.where(lane == r, cnts[bc // 16][bc % 16], v)
            l_idx = jnp.maximum(v, 1) - 1
            lvs.append(l_idx)
            lbs.append(l_idx * s_l + wid * s_bt
                       + _PERMUTATION_COUNT * (g * 16) + _PERMUTATION_COUNT * lane)

        lmin = jnp.min(jnp.minimum(jnp.minimum(lvs[0], lvs[1]),
                                   jnp.minimum(lvs[2], lvs[3])))
        lmax = jnp.max(jnp.maximum(jnp.maximum(lvs[0], lvs[1]),
                                   jnp.maximum(lvs[2], lvs[3])))
        uniform = lmin == lmax

        @pl.when(uniform)
        def _fast():
            base = lmin * s_l + wid * s_bt
            for dt in range(D // 8):
                pltpu.sync_copy(
                    emb_hbm.at[pl.ds(base + dt * s_dt, 8 * 128)],
                    blk_v.at[dt],
                )
            ev = jnp.where(lane < 8, lane * 2, lane * 2 - 16)

            def dt_body(dt, _):
                for dr in range(8):
                    vs = [blk_v[dt, pl.ds(dr * 128 + k * 16, 16)]
                          for k in range(8)]
                    for k in range(4):
                        g1 = _lane_gather(vs[2 * k], ev)
                        g2 = _lane_gather(vs[2 * k + 1], ev)
                        dst_v[dt, dr, pl.ds(k * 16, 16)] = (
                            jnp.where(lane < 8, g1, g2))
                return 0

            lax.fori_loop(0, D // 8, dt_body, 0)

        @pl.when(jnp.logical_not(uniform))
        def _slow():
            def idx_body(d, _):
                hi = (d >> 3) * s_dt + (d & 7) * 128
                for g in range(4):
                    idx_v[d, pl.ds(g * 16, 16)] = lbs[g] + hi
                return 0

            lax.fori_loop(0, D, idx_body, 0)

            def fire_body(d, _):
                pltpu.async_copy(
                    emb_hbm.at[idx_v.at[d]], dst_v.at[d >> 3, d & 7], sem
                )
                return 0

            def drain_body(d, _):
                pltpu.make_async_copy(
                    emb_hbm.at[idx_v.at[d]], dst_v.at[d >> 3, d & 7], sem
                ).wait()
                return 0

            lax.fori_loop(0, D, fire_body, 0)
            lax.fori_loop(0, D, drain_body, 0)
        pltpu.sync_copy(
            dst_v,
            out_hbm.at[:, pl.ds((wid >> 1) * 8, 8),
                       pl.ds((wid & 1) * rpw, rpw)],
        )

    return sc_kernel


def kernel(embeddings, labels, embeddings_mask, labels_mask):
    B, L, D = embeddings.shape
    O = len(range(0, B - 1, _PERMUTATION_COUNT))
    info = plsc.get_sparse_core_info()
    NC, NS = info.num_cores, info.num_subcores

    emb_phys = (
        embeddings.reshape(B // 128, 128, L, D // 8, 8)
        .transpose(2, 3, 0, 4, 1)
        .reshape(B * L * D)
    )
    cnt = (
        _tc_lengths(L, B)(labels_mask.T.view(jnp.int8))
        .reshape(8, B // 128, 128)
        .transpose(1, 0, 2)
        .reshape(8 * B)
    )

    out = _make_sc_kernel(B, L, D, O, NC, NS)(cnt, emb_phys)
    return (
        out.reshape(D // 8, O // 128, 8, 128)
        .transpose(1, 3, 0, 2)
        .reshape(O, 1, D)
    )
